# baseline (speedup 1.0000x reference)
import functools

import jax
import jax.numpy as jnp
from jax import lax
from jax.experimental import pallas as pl
from jax.experimental.pallas import tpu as pltpu
from jax.experimental.pallas import tpu_sc as plsc

_VEC = 16
_CS = 8
_NBUF = 3
_S_SC = 1024
_SB = 1024


def _sc_half(x, pe, s_count):
    B, S, D = x.shape
    info = plsc.get_sparse_core_info()
    nw = info.num_cores * info.num_subcores
    s_per_w = s_count // nw
    n_chunks = s_per_w // _CS
    assert D & (D - 1) == 0
    d_shift = D.bit_length() - 1
    mesh = plsc.VectorSubcoreMesh(core_axis_name="c", subcore_axis_name="s")

    @functools.partial(
        pl.kernel,
        mesh=mesh,
        out_type=jax.ShapeDtypeStruct((B, s_count, D), jnp.float32),
        scratch_types=(
            [pltpu.VMEM((_CS, D), jnp.float32) for _ in range(_NBUF)]
            + [pltpu.VMEM((B, _CS, D), jnp.float32) for _ in range(_NBUF)]
            + [pltpu.SemaphoreType.DMA for _ in range(2 * _NBUF)]
        ),
    )
    def sc_add(x_hbm, pe_hbm, out_hbm, *scratch):
        pe_bufs = scratch[:_NBUF]
        x_bufs = scratch[_NBUF : 2 * _NBUF]
        in_sems = scratch[2 * _NBUF : 3 * _NBUF]
        out_sems = scratch[3 * _NBUF : 4 * _NBUF]
        wid = lax.axis_index("s") * info.num_cores + lax.axis_index("c")
        s_base = wid * s_per_w

        def start_in(c, buf):
            s0 = s_base + c * _CS
            return [
                pltpu.async_copy(pe_hbm.at[pl.ds(s0, _CS), :], pe_bufs[buf], in_sems[buf]),
                pltpu.async_copy(x_hbm.at[:, pl.ds(s0, _CS), :], x_bufs[buf], in_sems[buf]),
            ]

        def start_out(c, buf):
            s0 = s_base + c * _CS
            return [
                pltpu.async_copy(x_bufs[buf], out_hbm.at[:, pl.ds(s0, _CS), :], out_sems[buf])
            ]

        handles_in = {0: start_in(0, 0)}
        handles_out = {}
        for c in range(n_chunks):
            buf = c % _NBUF
            for h in handles_in.pop(c):
                h.wait()
            if c + 1 < n_chunks:
                if c - (_NBUF - 1) in handles_out:
                    for h in handles_out.pop(c - (_NBUF - 1)):
                        h.wait()
                handles_in[c + 1] = start_in(c + 1, (c + 1) % _NBUF)

            pe_b = pe_bufs[buf]
            x_b = x_bufs[buf]

            @plsc.parallel_loop(0, _CS * D, step=_VEC, unroll=8)
            def _(i):
                r = lax.shift_right_logical(i, d_shift)
                col = pl.multiple_of(lax.bitwise_and(i, D - 1), _VEC)
                sl = pl.ds(col, _VEC)
                pv = pe_b[r, sl]
                for b in range(B):
                    x_b[b, r, sl] += pv

            handles_out[c] = start_out(c, buf)
        for c in sorted(handles_out):
            for h in handles_out[c]:
                h.wait()

    return sc_add(x, pe)


def _tc_add_kernel(x_ref, pe_ref, o_ref):
    o_ref[...] = x_ref[...] + pe_ref[...]


def _tc_half(x, pe, s_lo):
    B, S, D = x.shape
    off = s_lo // _SB
    grid = ((S - s_lo) // _SB, B)
    return pl.pallas_call(
        _tc_add_kernel,
        grid=grid,
        in_specs=[
            pl.BlockSpec((1, _SB, D), lambda j, i: (i, j + off, 0)),
            pl.BlockSpec((_SB, D), lambda j, i: (j + off, 0)),
        ],
        out_specs=pl.BlockSpec((1, _SB, D), lambda j, i: (i, j + off, 0)),
        out_shape=jax.ShapeDtypeStruct((B, S, D), x.dtype),
    )(x, pe)


def kernel(x, pe):
    sc_out = _sc_half(x, pe, _S_SC)
    tc_out = _tc_half(x, pe, _S_SC)
    return lax.dynamic_update_slice(tc_out, sc_out, (0, 0, 0))


# speedup vs baseline: 1.0316x; 1.0316x over previous
import functools

import jax
import jax.numpy as jnp
from jax import lax
from jax.experimental import pallas as pl
from jax.experimental.pallas import tpu as pltpu
from jax.experimental.pallas import tpu_sc as plsc

_VEC = 16
_CS = 8
_NBUF = 3


def kernel(x, pe):
    B, S, D = x.shape
    info = plsc.get_sparse_core_info()
    nw = info.num_cores * info.num_subcores
    s_per_w = S // nw
    n_chunks = s_per_w // _CS
    assert D & (D - 1) == 0
    d_shift = D.bit_length() - 1
    mesh = plsc.VectorSubcoreMesh(core_axis_name="c", subcore_axis_name="s")

    @functools.partial(
        pl.kernel,
        mesh=mesh,
        out_type=jax.ShapeDtypeStruct((B, S, D), jnp.float32),
        scratch_types=(
            [pltpu.VMEM((_CS, D), jnp.float32) for _ in range(_NBUF)]
            + [pltpu.VMEM((B, _CS, D), jnp.float32) for _ in range(_NBUF)]
            + [pltpu.SemaphoreType.DMA for _ in range(2 * _NBUF)]
        ),
    )
    def sc_add(x_hbm, pe_hbm, out_hbm, *scratch):
        pe_bufs = scratch[:_NBUF]
        x_bufs = scratch[_NBUF : 2 * _NBUF]
        in_sems = scratch[2 * _NBUF : 3 * _NBUF]
        out_sems = scratch[3 * _NBUF : 4 * _NBUF]
        wid = lax.axis_index("s") * info.num_cores + lax.axis_index("c")
        s_base = wid * s_per_w

        def start_in(c, buf):
            s0 = s_base + c * _CS
            return [
                pltpu.async_copy(pe_hbm.at[pl.ds(s0, _CS), :], pe_bufs[buf], in_sems[buf]),
                pltpu.async_copy(x_hbm.at[:, pl.ds(s0, _CS), :], x_bufs[buf], in_sems[buf]),
            ]

        def start_out(c, buf):
            s0 = s_base + c * _CS
            return [
                pltpu.async_copy(x_bufs[buf], out_hbm.at[:, pl.ds(s0, _CS), :], out_sems[buf])
            ]

        handles_in = {0: start_in(0, 0)}
        handles_out = {}
        for c in range(n_chunks):
            buf = c % _NBUF
            for h in handles_in.pop(c):
                h.wait()
            if c + 1 < n_chunks:
                if c - (_NBUF - 1) in handles_out:
                    for h in handles_out.pop(c - (_NBUF - 1)):
                        h.wait()
                handles_in[c + 1] = start_in(c + 1, (c + 1) % _NBUF)

            pe_b = pe_bufs[buf]
            x_b = x_bufs[buf]

            @plsc.parallel_loop(0, _CS * D, step=_VEC, unroll=8)
            def _(i):
                r = lax.shift_right_logical(i, d_shift)
                col = pl.multiple_of(lax.bitwise_and(i, D - 1), _VEC)
                sl = pl.ds(col, _VEC)
                pv = pe_b[r, sl]
                for b in range(B):
                    x_b[b, r, sl] += pv

            handles_out[c] = start_out(c, buf)
        for c in sorted(handles_out):
            for h in handles_out[c]:
                h.wait()

    return sc_add(x, pe)
